# baseline (speedup 1.0000x reference)
import functools

import jax
import jax.numpy as jnp
from jax import lax
from jax.experimental import pallas as pl
from jax.experimental.pallas import tpu as pltpu
from jax.experimental.pallas import tpu_sc as plsc

BS = 4
NP = 8192
NV = 5023
NVP = 5120
TN = 2048
NSTEPS = BS * NP // TN
NROUND = 4
HSTEPS = NSTEPS // NROUND
HPTS = HSTEPS * TN
STEPS_PER_B = NP // TN
DS, DP, DL = 150, 108, 5
DT = 384
NW = 32
BPW = HPTS // NW
CH = 128
NCH = BPW // CH


def _prep_body(bet_ref, fs_ref, vt_ref, w_ref):
    bet = bet_ref[...]
    c0 = jnp.dot(bet, fs_ref[0], precision=lax.Precision.HIGHEST) + vt_ref[0][None, :]
    c1 = jnp.dot(bet, fs_ref[1], precision=lax.Precision.HIGHEST) + vt_ref[1][None, :]
    c2 = jnp.dot(bet, fs_ref[2], precision=lax.Precision.HIGHEST) + vt_ref[2][None, :]
    v2 = c0 * c0 + c1 * c1 + c2 * c2
    z = jnp.zeros_like(v2)
    w_ref[:, :, 0:NV] = jnp.stack(
        [-2.0 * c0, -2.0 * c1, -2.0 * c2, v2, z, z, z, z], axis=1)
    zp = jnp.zeros((BS, 1, NVP - NV), jnp.float32)
    w_ref[:, :, NV:] = jnp.concatenate(
        [zp, zp, zp, jnp.full((BS, 1, NVP - NV), 1.0e12, jnp.float32),
         zp, zp, zp, zp], axis=1)


def _build_w(betas, fs_t, vt_t):
    return pl.pallas_call(
        _prep_body,
        out_shape=jax.ShapeDtypeStruct((BS, 8, NVP), jnp.float32),
    )(betas, fs_t, vt_t)


def _knn_body(p_ref, w_ref, idx_ref, dm_ref):
    p = p_ref[...]
    px = p[:, 0:1]
    py = p[:, 1:2]
    pz = p[:, 2:3]
    w0 = w_ref[0, 0:1, :]
    w1 = w_ref[0, 1:2, :]
    w2 = w_ref[0, 2:3, :]
    v2 = w_ref[0, 3:4, :]
    d2p = px * w0 + py * w1 + pz * w2 + v2
    vidx = lax.broadcasted_iota(jnp.int32, (TN, NVP), 1)
    packed = lax.bitcast_convert_type(
        (lax.bitcast_convert_type(d2p, jnp.int32) & ~8191) | vidx, jnp.float32)
    mn = jnp.min(packed, axis=-1)
    mb = lax.bitcast_convert_type(mn, jnp.int32)
    amin = mb & 8191
    dmin = lax.bitcast_convert_type(mb & ~8191, jnp.float32)
    p2 = px * px + py * py + pz * pz
    idx_ref[0, 0, :] = amin
    dm_ref[0, 0, :] = dmin + p2[:, 0]


def _knn(pts, w, base):
    return pl.pallas_call(
        _knn_body,
        grid=(HSTEPS,),
        in_specs=[
            pl.BlockSpec((TN, 3), lambda i: (i + base, 0)),
            pl.BlockSpec((1, 8, NVP), lambda i: ((i + base) // STEPS_PER_B, 0, 0)),
        ],
        out_specs=[
            pl.BlockSpec((1, 1, TN), lambda i: (i, 0, 0)),
            pl.BlockSpec((1, 1, TN), lambda i: (i, 0, 0)),
        ],
        out_shape=[
            jax.ShapeDtypeStruct((HSTEPS, 1, TN), jnp.int32),
            jax.ShapeDtypeStruct((HSTEPS, 1, TN), jnp.float32),
        ],
    )(pts, w)


@functools.cache
def _sc_gather_fn():
    mesh = plsc.VectorSubcoreMesh(core_axis_name="c", subcore_axis_name="s")

    @functools.partial(
        pl.kernel,
        mesh=mesh,
        out_type=jax.ShapeDtypeStruct((HPTS, DT), jnp.float32),
        scratch_types=[
            pltpu.VMEM((NCH, CH), jnp.int32),
            pltpu.VMEM((CH, DT), jnp.float32),
            pltpu.VMEM((CH, DT), jnp.float32),
            pltpu.SemaphoreType.DMA,
            pltpu.SemaphoreType.DMA,
        ],
    )
    def _sc_gather(t_hbm, idx_hbm, out_hbm, idx_v, buf0, buf1, sem0, sem1):
        wid = lax.axis_index("s") * 2 + lax.axis_index("c")
        pltpu.sync_copy(idx_hbm.at[wid], idx_v)
        bufs = (buf0, buf1)
        sems = (sem0, sem1)
        handles = [None] * NCH
        handles[0] = pltpu.async_copy(t_hbm.at[idx_v.at[0]], bufs[0], sems[0])
        for c in range(NCH):
            if c + 1 < NCH:
                handles[c + 1] = pltpu.async_copy(
                    t_hbm.at[idx_v.at[c + 1]], bufs[(c + 1) % 2], sems[(c + 1) % 2])
            handles[c].wait()
            pltpu.sync_copy(bufs[c % 2],
                            out_hbm.at[pl.ds(wid * BPW + c * CH, CH)])

    return _sc_gather


def _loss_body(as_ref, ap_ref, al_ref, g_ref, dm_ref, out_ref, acc_ref):
    step = pl.program_id(0)

    @pl.when(step == 0)
    def _():
        acc_ref[0] = 0.0
        acc_ref[1] = 0.0

    g = g_ref[...]
    m = (dm_ref[0, 0, :] < 0.1).astype(jnp.float32)
    mrow = m[None, :]
    mcol = jnp.transpose(mrow)

    lane = lax.broadcasted_iota(jnp.int32, (1, DT), 1)
    wl = jnp.where(lane < DS + DP, 100.0,
                   jnp.where(lane < DS + DP + DL, 0.2, 0.0))
    sum_b2 = jnp.sum(g * g * wl * mcol)

    m100 = mrow * 100.0
    a_s = as_ref[...]
    a_p = ap_ref[...]
    a_l = al_ref[...]
    am_all = jnp.concatenate(
        [a_s[0] * m100, a_s[1] * m100, a_s[2] * m100,
         a_p[0] * m100, a_p[1] * m100, a_p[2] * m100,
         a_l * (mrow * 0.2)], axis=0)
    nd = DS + DP + DL
    pr = jnp.dot(am_all, g[:, 0:nd], precision=lax.Precision.HIGHEST)
    eye = (lax.broadcasted_iota(jnp.int32, (nd, nd), 0)
           == lax.broadcasted_iota(jnp.int32, (nd, nd), 1))
    cross = jnp.sum(jnp.where(eye, pr, 0.0))
    m3 = m[None, None, :]
    a2s = jnp.sum(a_s * a_s * m3)
    a2p = jnp.sum(a_p * a_p * m3)
    a2l = jnp.sum(a_l * a_l * mrow)

    step_num = (100.0 * a2s + 100.0 * a2p + 0.2 * a2l - 2.0 * cross + sum_b2)
    acc_ref[0] = acc_ref[0] + step_num
    acc_ref[1] = acc_ref[1] + jnp.sum(m)

    @pl.when(step == HSTEPS - 1)
    def _():
        out_ref[...] = jnp.concatenate(
            [jnp.full((1, 1), acc_ref[0], jnp.float32),
             jnp.full((1, 1), acc_ref[1], jnp.float32)], axis=1)


def _loss(a_s, a_p, a_l, g, dmin, base):
    return pl.pallas_call(
        _loss_body,
        grid=(HSTEPS,),
        in_specs=[
            pl.BlockSpec((3, 50, TN), lambda i: (0, 0, i + base)),
            pl.BlockSpec((3, 36, TN), lambda i: (0, 0, i + base)),
            pl.BlockSpec((DL, TN), lambda i: (0, i + base)),
            pl.BlockSpec((TN, DT), lambda i: (i, 0)),
            pl.BlockSpec((1, 1, TN), lambda i: (i, 0, 0)),
        ],
        out_specs=pl.BlockSpec((1, 2), lambda i: (0, 0)),
        out_shape=jax.ShapeDtypeStruct((1, 2), jnp.float32),
        scratch_shapes=[pltpu.SMEM((2,), jnp.float32)],
    )(a_s, a_p, a_l, g, dmin)


def _comb_body(*refs):
    out_ref = refs[-1]
    tot = refs[0][...]
    for r in refs[1:-1]:
        tot = tot + r[...]
    num = tot[0, 0]
    cnt = tot[0, 1]
    loss = num / jnp.maximum(cnt, 1.0)
    out_ref[...] = jnp.full((1, 1), jnp.where(cnt == 0.0, 0.0, loss),
                            jnp.float32)


def _combine(parts):
    return pl.pallas_call(
        _comb_body,
        out_shape=jax.ShapeDtypeStruct((1, 1), jnp.float32),
    )(*parts)


def kernel(shapedirs, posedirs, lbs_weights, pts_c, flame_params,
           flame_shapedirs, flame_posedirs, flame_lbs_weights, v_template,
           canonical_exp):
    betas = jnp.concatenate(
        [flame_params[:, -150:-50],
         jnp.broadcast_to(canonical_exp, (BS, canonical_exp.shape[0]))], axis=1)

    fs_t = jnp.transpose(flame_shapedirs, (1, 2, 0))
    vt_t = v_template.T
    w = _build_w(betas, fs_t, vt_t)

    b_s = flame_shapedirs[:, :, -50:].reshape(NV, DS)
    b_p = jnp.transpose(flame_posedirs.reshape(36, NV, 3), (1, 2, 0)).reshape(NV, DP)
    b_l = flame_lbs_weights
    table = jnp.concatenate(
        [b_s, b_p, b_l, jnp.zeros((NV, DT - DS - DP - DL), jnp.float32)], axis=1)

    a_s = jnp.transpose(shapedirs, (1, 2, 0))
    a_p = jnp.transpose(posedirs, (2, 1, 0))
    a_l = jnp.transpose(lbs_weights.reshape(BS * NP, DL), (1, 0))

    gather = _sc_gather_fn()
    gs = []
    dms = []
    for r in range(NROUND):
        idx_r, dm_r = _knn(pts_c, w, r * HSTEPS)
        gs.append(gather(table, idx_r.reshape(NW, NCH, CH)))
        dms.append(dm_r)
    parts = [_loss(a_s, a_p, a_l, gs[r], dms[r], r * HSTEPS)
             for r in range(NROUND)]
    out = _combine(parts)
    return out[0, 0]


# speedup vs baseline: 1.0278x; 1.0278x over previous
import functools

import jax
import jax.numpy as jnp
from jax import lax
from jax.experimental import pallas as pl
from jax.experimental.pallas import tpu as pltpu
from jax.experimental.pallas import tpu_sc as plsc

BS = 4
NP = 8192
NV = 5023
NVP = 5120
TN = 1024
NSTEPS = BS * NP // TN
NROUND = 4
HSTEPS = NSTEPS // NROUND
HPTS = HSTEPS * TN
STEPS_PER_B = NP // TN
DS, DP, DL = 150, 108, 5
DT = 384
NW = 32
BPW = HPTS // NW
CH = 128
NCH = BPW // CH


def _prep_body(bet_ref, fs_ref, vt_ref, w_ref):
    bet = bet_ref[...]
    c0 = jnp.dot(bet, fs_ref[0], precision=lax.Precision.HIGHEST) + vt_ref[0][None, :]
    c1 = jnp.dot(bet, fs_ref[1], precision=lax.Precision.HIGHEST) + vt_ref[1][None, :]
    c2 = jnp.dot(bet, fs_ref[2], precision=lax.Precision.HIGHEST) + vt_ref[2][None, :]
    v2 = c0 * c0 + c1 * c1 + c2 * c2
    z = jnp.zeros_like(v2)
    w_ref[:, :, 0:NV] = jnp.stack(
        [-2.0 * c0, -2.0 * c1, -2.0 * c2, v2, z, z, z, z], axis=1)
    zp = jnp.zeros((BS, 1, NVP - NV), jnp.float32)
    w_ref[:, :, NV:] = jnp.concatenate(
        [zp, zp, zp, jnp.full((BS, 1, NVP - NV), 1.0e12, jnp.float32),
         zp, zp, zp, zp], axis=1)


def _build_w(betas, fs_t, vt_t):
    return pl.pallas_call(
        _prep_body,
        out_shape=jax.ShapeDtypeStruct((BS, 8, NVP), jnp.float32),
    )(betas, fs_t, vt_t)


def _knn_body(p_ref, w_ref, idx_ref, dm_ref):
    p = p_ref[...]
    px = p[:, 0:1]
    py = p[:, 1:2]
    pz = p[:, 2:3]
    w0 = w_ref[0, 0:1, :]
    w1 = w_ref[0, 1:2, :]
    w2 = w_ref[0, 2:3, :]
    v2 = w_ref[0, 3:4, :]
    d2p = px * w0 + py * w1 + pz * w2 + v2
    vidx = lax.broadcasted_iota(jnp.int32, (TN, NVP), 1)
    packed = lax.bitcast_convert_type(
        (lax.bitcast_convert_type(d2p, jnp.int32) & ~8191) | vidx, jnp.float32)
    mn = jnp.min(packed, axis=-1)
    mb = lax.bitcast_convert_type(mn, jnp.int32)
    amin = mb & 8191
    dmin = lax.bitcast_convert_type(mb & ~8191, jnp.float32)
    p2 = px * px + py * py + pz * pz
    idx_ref[0, 0, :] = amin
    dm_ref[0, 0, :] = dmin + p2[:, 0]


def _knn(pts, w, base):
    return pl.pallas_call(
        _knn_body,
        grid=(HSTEPS,),
        in_specs=[
            pl.BlockSpec((TN, 3), lambda i: (i + base, 0)),
            pl.BlockSpec((1, 8, NVP), lambda i: ((i + base) // STEPS_PER_B, 0, 0)),
        ],
        out_specs=[
            pl.BlockSpec((1, 1, TN), lambda i: (i, 0, 0)),
            pl.BlockSpec((1, 1, TN), lambda i: (i, 0, 0)),
        ],
        out_shape=[
            jax.ShapeDtypeStruct((HSTEPS, 1, TN), jnp.int32),
            jax.ShapeDtypeStruct((HSTEPS, 1, TN), jnp.float32),
        ],
    )(pts, w)


@functools.cache
def _sc_gather_fn():
    mesh = plsc.VectorSubcoreMesh(core_axis_name="c", subcore_axis_name="s")

    @functools.partial(
        pl.kernel,
        mesh=mesh,
        out_type=jax.ShapeDtypeStruct((HPTS, DT), jnp.float32),
        scratch_types=[
            pltpu.VMEM((NCH, CH), jnp.int32),
            pltpu.VMEM((CH, DT), jnp.float32),
            pltpu.VMEM((CH, DT), jnp.float32),
            pltpu.SemaphoreType.DMA,
            pltpu.SemaphoreType.DMA,
        ],
    )
    def _sc_gather(t_hbm, idx_hbm, out_hbm, idx_v, buf0, buf1, sem0, sem1):
        wid = lax.axis_index("s") * 2 + lax.axis_index("c")
        pltpu.sync_copy(idx_hbm.at[wid], idx_v)
        bufs = (buf0, buf1)
        sems = (sem0, sem1)
        handles = [None] * NCH
        handles[0] = pltpu.async_copy(t_hbm.at[idx_v.at[0]], bufs[0], sems[0])
        for c in range(NCH):
            if c + 1 < NCH:
                handles[c + 1] = pltpu.async_copy(
                    t_hbm.at[idx_v.at[c + 1]], bufs[(c + 1) % 2], sems[(c + 1) % 2])
            handles[c].wait()
            pltpu.sync_copy(bufs[c % 2],
                            out_hbm.at[pl.ds(wid * BPW + c * CH, CH)])

    return _sc_gather


def _loss_body(as_ref, ap_ref, al_ref, g_ref, dm_ref, out_ref, acc_ref):
    step = pl.program_id(0)

    @pl.when(step == 0)
    def _():
        acc_ref[0] = 0.0
        acc_ref[1] = 0.0

    g = g_ref[...]
    m = (dm_ref[0, 0, :] < 0.1).astype(jnp.float32)
    mrow = m[None, :]
    mcol = jnp.transpose(mrow)

    lane = lax.broadcasted_iota(jnp.int32, (1, DT), 1)
    wl = jnp.where(lane < DS + DP, 100.0,
                   jnp.where(lane < DS + DP + DL, 0.2, 0.0))
    sum_b2 = jnp.sum(g * g * wl * mcol)

    m100 = mrow * 100.0
    a_s = as_ref[...]
    a_p = ap_ref[...]
    a_l = al_ref[...]
    am_all = jnp.concatenate(
        [a_s[0] * m100, a_s[1] * m100, a_s[2] * m100,
         a_p[0] * m100, a_p[1] * m100, a_p[2] * m100,
         a_l * (mrow * 0.2)], axis=0)
    nd = DS + DP + DL
    pr = jnp.dot(am_all, g[:, 0:nd], precision=lax.Precision.HIGHEST)
    eye = (lax.broadcasted_iota(jnp.int32, (nd, nd), 0)
           == lax.broadcasted_iota(jnp.int32, (nd, nd), 1))
    cross = jnp.sum(jnp.where(eye, pr, 0.0))
    m3 = m[None, None, :]
    a2s = jnp.sum(a_s * a_s * m3)
    a2p = jnp.sum(a_p * a_p * m3)
    a2l = jnp.sum(a_l * a_l * mrow)

    step_num = (100.0 * a2s + 100.0 * a2p + 0.2 * a2l - 2.0 * cross + sum_b2)
    acc_ref[0] = acc_ref[0] + step_num
    acc_ref[1] = acc_ref[1] + jnp.sum(m)

    @pl.when(step == HSTEPS - 1)
    def _():
        out_ref[...] = jnp.concatenate(
            [jnp.full((1, 1), acc_ref[0], jnp.float32),
             jnp.full((1, 1), acc_ref[1], jnp.float32)], axis=1)


def _loss(a_s, a_p, a_l, g, dmin, base):
    return pl.pallas_call(
        _loss_body,
        grid=(HSTEPS,),
        in_specs=[
            pl.BlockSpec((3, 50, TN), lambda i: (0, 0, i + base)),
            pl.BlockSpec((3, 36, TN), lambda i: (0, 0, i + base)),
            pl.BlockSpec((DL, TN), lambda i: (0, i + base)),
            pl.BlockSpec((TN, DT), lambda i: (i, 0)),
            pl.BlockSpec((1, 1, TN), lambda i: (i, 0, 0)),
        ],
        out_specs=pl.BlockSpec((1, 2), lambda i: (0, 0)),
        out_shape=jax.ShapeDtypeStruct((1, 2), jnp.float32),
        scratch_shapes=[pltpu.SMEM((2,), jnp.float32)],
    )(a_s, a_p, a_l, g, dmin)


def _comb_body(*refs):
    out_ref = refs[-1]
    tot = refs[0][...]
    for r in refs[1:-1]:
        tot = tot + r[...]
    num = tot[0, 0]
    cnt = tot[0, 1]
    loss = num / jnp.maximum(cnt, 1.0)
    out_ref[...] = jnp.full((1, 1), jnp.where(cnt == 0.0, 0.0, loss),
                            jnp.float32)


def _combine(parts):
    return pl.pallas_call(
        _comb_body,
        out_shape=jax.ShapeDtypeStruct((1, 1), jnp.float32),
    )(*parts)


def kernel(shapedirs, posedirs, lbs_weights, pts_c, flame_params,
           flame_shapedirs, flame_posedirs, flame_lbs_weights, v_template,
           canonical_exp):
    betas = jnp.concatenate(
        [flame_params[:, -150:-50],
         jnp.broadcast_to(canonical_exp, (BS, canonical_exp.shape[0]))], axis=1)

    fs_t = jnp.transpose(flame_shapedirs, (1, 2, 0))
    vt_t = v_template.T
    w = _build_w(betas, fs_t, vt_t)

    b_s = flame_shapedirs[:, :, -50:].reshape(NV, DS)
    b_p = jnp.transpose(flame_posedirs.reshape(36, NV, 3), (1, 2, 0)).reshape(NV, DP)
    b_l = flame_lbs_weights
    table = jnp.concatenate(
        [b_s, b_p, b_l, jnp.zeros((NV, DT - DS - DP - DL), jnp.float32)], axis=1)

    a_s = jnp.transpose(shapedirs, (1, 2, 0))
    a_p = jnp.transpose(posedirs, (2, 1, 0))
    a_l = jnp.transpose(lbs_weights.reshape(BS * NP, DL), (1, 0))

    gather = _sc_gather_fn()
    gs = []
    dms = []
    for r in range(NROUND):
        idx_r, dm_r = _knn(pts_c, w, r * HSTEPS)
        gs.append(gather(table, idx_r.reshape(NW, NCH, CH)))
        dms.append(dm_r)
    parts = [_loss(a_s, a_p, a_l, gs[r], dms[r], r * HSTEPS)
             for r in range(NROUND)]
    out = _combine(parts)
    return out[0, 0]
